# baseline (speedup 1.0000x reference)
import functools

import jax
import jax.numpy as jnp
from jax import lax
from jax.experimental import pallas as pl
from jax.experimental.pallas import tpu as pltpu
from jax.experimental.pallas import tpu_sc as plsc

N_NODES = 50000
N_EDGES = 800000
ATOM_IN = 100
HID = 64
BOND = 41
N_CONV = 3

NC = 2
NS = 16
L = 16
HALF = HID // NC

EPT = N_EDGES // NS
C = 80
NCHUNK = EPT // C
DRN = 80
NDC = N_NODES // DRN
KDC = -(-NDC // NS)


_NNB = 25


def _lin0_body(x_ref, w_ref, b_ref, o_ref):
    acc = jnp.dot(x_ref[...], w_ref[0], preferred_element_type=jnp.float32)
    o_ref[...] = jnp.maximum(acc + b_ref[0], 0.0)


def _lin0(n_feat, W0s, b0s):
    BN = N_NODES // _NNB
    return pl.pallas_call(
        _lin0_body,
        grid=(NC, _NNB),
        in_specs=[
            pl.BlockSpec((BN, ATOM_IN), lambda h, i: (i, 0)),
            pl.BlockSpec((1, ATOM_IN, HALF), lambda h, i: (h, 0, 0)),
            pl.BlockSpec((1, 1, HALF), lambda h, i: (h, 0, 0)),
        ],
        out_specs=pl.BlockSpec((BN, HALF), lambda h, i: (h * _NNB + i, 0)),
        out_shape=jax.ShapeDtypeStruct((NC * N_NODES, HALF), jnp.float32),
    )(n_feat, W0s, b0s)


def _eterm_body(e_ref, w_ref, b_ref, o_ref):
    acc = jnp.dot(e_ref[...], w_ref[0], preferred_element_type=jnp.float32)
    o_ref[0] = acc + b_ref[0]


def _eterm(edge_attr, Wes, bcs):
    BE = 4000
    return pl.pallas_call(
        _eterm_body,
        grid=(NC, N_EDGES // BE),
        in_specs=[
            pl.BlockSpec((BE, BOND), lambda h, i: (i, 0)),
            pl.BlockSpec((1, BOND, HALF), lambda h, i: (h, 0, 0)),
            pl.BlockSpec((1, 1, HALF), lambda h, i: (h, 0, 0)),
        ],
        out_specs=pl.BlockSpec((1, BE, HALF), lambda h, i: (h, i, 0)),
        out_shape=jax.ShapeDtypeStruct((NC, N_EDGES, HALF), jnp.float32),
    )(edge_attr, Wes, bcs)


def _ab_body(x0_ref, x1_ref, w_ref, a_ref, b_ref):
    acc = jnp.dot(x0_ref[...], w_ref[0, 0], preferred_element_type=jnp.float32)
    acc = acc + jnp.dot(x1_ref[...], w_ref[0, 1],
                        preferred_element_type=jnp.float32)
    a_ref[...] = acc[:, :HALF]
    b_ref[...] = acc[:, HALF:]


def _ab(out, W4):
    BN = N_NODES // _NNB
    spec_flat = pl.BlockSpec((BN, HALF), lambda h, i: (h * _NNB + i, 0))
    return pl.pallas_call(
        _ab_body,
        grid=(NC, _NNB),
        in_specs=[
            pl.BlockSpec((BN, HALF), lambda h, i: (i, 0)),
            pl.BlockSpec((BN, HALF), lambda h, i: (_NNB + i, 0)),
            pl.BlockSpec((1, 2, HALF, HID), lambda h, i: (h, 0, 0, 0)),
        ],
        out_specs=[spec_flat, spec_flat],
        out_shape=[
            jax.ShapeDtypeStruct((NC * N_NODES, HALF), jnp.float32),
            jax.ShapeDtypeStruct((NC * N_NODES, HALF), jnp.float32),
        ],
    )(out, out, W4)


_HB = 5000
_HNB = NC * N_NODES // _HB


def _head_body(x_ref, w_ref, b_ref, o_ref, sacc):
    i = pl.program_id(0)

    @pl.when(i == 0)
    def _():
        sacc[...] = jnp.zeros_like(sacc)

    s = jnp.sum(x_ref[...], axis=0, keepdims=True)

    @pl.when(i < _HNB // 2)
    def _():
        sacc[:, :HALF] += s

    @pl.when(i >= _HNB // 2)
    def _():
        sacc[:, HALF:] += s

    @pl.when(i == _HNB - 1)
    def _():
        rep = jnp.maximum(sacc[...], 0.0)
        v = jnp.sum(rep * w_ref[...], keepdims=True) + b_ref[...]
        o_ref[...] = jnp.maximum(v, 0.0)


def _head(out, W2t, b2):
    return pl.pallas_call(
        _head_body,
        grid=(_HNB,),
        in_specs=[
            pl.BlockSpec((_HB, HALF), lambda i: (i, 0)),
            pl.BlockSpec((1, HID), lambda i: (0, 0)),
            pl.BlockSpec((1, 1), lambda i: (0, 0)),
        ],
        out_specs=pl.BlockSpec((1, 1), lambda i: (0, 0)),
        out_shape=jax.ShapeDtypeStruct((1, 1), jnp.float32),
        scratch_shapes=[pltpu.VMEM((1, HID), jnp.float32)],
    )(out, W2t, b2)


def _edge_body(a_hbm, b_hbm, et_hbm, src_hbm, dst_hbm, out_hbm,
               sidx, didx, didx2, didxs, arow, brow, erow, mrow, zbuf, acc,
               idx_sem, gat_sem, sc_sem):
    cid = lax.axis_index("c")
    tid = lax.axis_index("s")
    e0 = tid * EPT
    coff = cid * N_NODES

    zeros16 = jnp.zeros((L,), jnp.float32)

    def zrow(r, carry):
        for j in range(HALF // L):
            zbuf[r, pl.ds(j * L, L)] = zeros16
        return carry

    lax.fori_loop(0, DRN, zrow, 0)
    for k in range(KDC):
        m = tid + NS * k

        @pl.when(m < NDC)
        def _():
            pltpu.sync_copy(zbuf, acc.at[pl.ds(m * DRN, DRN)])

    plsc.subcore_barrier()

    def issue_idx(i, bi):
        off = e0 + i * C
        pltpu.async_copy(src_hbm.at[pl.ds(off, C)], sidx.at[bi], idx_sem)
        pltpu.async_copy(dst_hbm.at[pl.ds(off, C)], didx.at[bi], idx_sem)

    def wait_idx(bi):
        pltpu.make_async_copy(src_hbm.at[pl.ds(0, C)], sidx.at[bi], idx_sem).wait()
        pltpu.make_async_copy(dst_hbm.at[pl.ds(0, C)], didx.at[bi], idx_sem).wait()

    def adjust_idx(bi):
        for j in range(C // L):
            s = pl.ds(j * L, L)
            sidx[bi, s] = sidx[bi, s] + coff
            didx2[bi, s] = didx[bi, s] + coff

    def issue_gather(i, bi):
        off = e0 + i * C
        pltpu.async_copy(a_hbm.at[sidx.at[bi]], arow.at[bi], gat_sem)
        pltpu.async_copy(b_hbm.at[didx2.at[bi]], brow.at[bi], gat_sem)
        pltpu.async_copy(et_hbm.at[cid, pl.ds(off, C)], erow.at[bi], gat_sem)

    def wait_gather(bi):
        pltpu.make_async_copy(a_hbm.at[sidx.at[bi]], arow.at[bi], gat_sem).wait()
        pltpu.make_async_copy(b_hbm.at[didx2.at[bi]], brow.at[bi], gat_sem).wait()
        pltpu.make_async_copy(et_hbm.at[cid, pl.ds(0, C)], erow.at[bi],
                              gat_sem).wait()

    def copy_scatter_idx(bi):
        for j in range(C // L):
            s = pl.ds(j * L, L)
            didxs[bi, s] = didx[bi, s]

    def issue_scatter(bi):
        pltpu.async_copy(mrow.at[bi], acc.at[didxs.at[bi]], sc_sem, add=True)

    def wait_scatter(bi):
        pltpu.make_async_copy(mrow.at[bi], acc.at[didxs.at[bi]], sc_sem).wait()

    CG = 16

    def compute(bi):
        def cgrp(g, carry):
            base = g * CG
            for r in range(CG):
                row = base + r
                for j in range(HALF // L):
                    s = pl.ds(j * L, L)
                    z = arow[bi, row, s] + brow[bi, row, s] + erow[bi, row, s]
                    lk = jnp.maximum(z, z * 0.01)
                    mrow[bi, row, s] = lk / (1.0 + jnp.exp(-z))
            return carry

        lax.fori_loop(0, C // CG, cgrp, 0)

    def chunk_step(i, bi):
        wait_gather(bi)

        @pl.when(i >= 2)
        def _():
            wait_scatter(bi)

        copy_scatter_idx(bi)

        @pl.when(i + 2 < NCHUNK)
        def _():
            issue_idx(i + 2, bi)

        compute(bi)
        issue_scatter(bi)

        @pl.when(i + 2 < NCHUNK)
        def _():
            wait_idx(bi)
            adjust_idx(bi)
            issue_gather(i + 2, bi)

    issue_idx(0, 0)
    issue_idx(1, 1)
    wait_idx(0)
    adjust_idx(0)
    issue_gather(0, 0)
    wait_idx(1)
    adjust_idx(1)
    issue_gather(1, 1)

    def pair_body(k, carry):
        i0 = 2 * k
        chunk_step(i0, 0)
        chunk_step(i0 + 1, 1)
        return carry

    lax.fori_loop(0, NCHUNK // 2, pair_body, 0)
    if NCHUNK % 2:
        chunk_step(NCHUNK - 1, 0)

    wait_scatter(1 - (NCHUNK % 2))
    wait_scatter(NCHUNK % 2)

    plsc.subcore_barrier()
    for k in range(KDC):
        m = tid + NS * k

        @pl.when(m < NDC)
        def _():
            pltpu.sync_copy(acc.at[pl.ds(m * DRN, DRN)],
                            out_hbm.at[pl.ds(coff + m * DRN, DRN)])


_edge_kernel = functools.partial(
    pl.kernel,
    out_type=jax.ShapeDtypeStruct((NC * N_NODES, HALF), jnp.float32),
    mesh=plsc.VectorSubcoreMesh(core_axis_name="c", subcore_axis_name="s",
                                num_cores=NC, num_subcores=NS),
    scratch_types=[
        pltpu.VMEM((2, C), jnp.int32),
        pltpu.VMEM((2, C), jnp.int32),
        pltpu.VMEM((2, C), jnp.int32),
        pltpu.VMEM((2, C), jnp.int32),
        pltpu.VMEM((2, C, HALF), jnp.float32),
        pltpu.VMEM((2, C, HALF), jnp.float32),
        pltpu.VMEM((2, C, HALF), jnp.float32),
        pltpu.VMEM((2, C, HALF), jnp.float32),
        pltpu.VMEM((DRN, HALF), jnp.float32),
        pltpu.VMEM_SHARED((N_NODES, HALF), jnp.float32),
        pltpu.SemaphoreType.DMA,
        pltpu.SemaphoreType.DMA,
        pltpu.SemaphoreType.DMA,
    ],
    compiler_params=pltpu.CompilerParams(use_tc_tiling_on_sc=False,
                                         internal_scratch_in_bytes=0),
)(_edge_body)


def kernel(n_feat, edge_index, edge_attr, W0, b0, Wc, bc, W2, b2):
    src = edge_index[0].astype(jnp.int32)
    dst = edge_index[1].astype(jnp.int32)
    Ws = Wc[:HID]
    Wd = Wc[HID:2 * HID]
    We = Wc[2 * HID:]

    W0s = W0.reshape(ATOM_IN, NC, HALF).transpose(1, 0, 2)
    b0s = b0.reshape(NC, 1, HALF)
    Wes = We.reshape(BOND, NC, HALF).transpose(1, 0, 2)
    bcs = bc.reshape(NC, 1, HALF)
    W4 = jnp.stack([
        jnp.stack([
            jnp.concatenate([Ws[k * HALF:(k + 1) * HALF, h * HALF:(h + 1) * HALF],
                             Wd[k * HALF:(k + 1) * HALF, h * HALF:(h + 1) * HALF]],
                            axis=1)
            for k in range(NC)])
        for h in range(NC)])

    out = _lin0(n_feat, W0s, b0s)
    et = _eterm(edge_attr, Wes, bcs)
    for _ in range(N_CONV):
        A, B = _ab(out, W4)
        out = _edge_kernel(A, B, et, src, dst)
    return _head(out, W2.reshape(1, HID), b2.reshape(1, 1))


# speedup vs baseline: 1.1726x; 1.1726x over previous
import functools

import jax
import jax.numpy as jnp
from jax import lax
from jax.experimental import pallas as pl
from jax.experimental.pallas import tpu as pltpu
from jax.experimental.pallas import tpu_sc as plsc

N_NODES = 50000
N_EDGES = 800000
ATOM_IN = 100
HID = 64
BOND = 41
N_CONV = 3

NC = 2
NS = 16
L = 16
HALF = HID // NC

EPT = N_EDGES // NS
C = 80
NCHUNK = EPT // C
DRN = 80
NDC = N_NODES // DRN
KDC = -(-NDC // NS)


_NNB = 25


def _lin0_body(x_ref, w_ref, b_ref, o_ref):
    acc = jnp.dot(x_ref[...], w_ref[0], preferred_element_type=jnp.float32)
    o_ref[...] = jnp.maximum(acc + b_ref[0], 0.0)


def _lin0(n_feat, W0s, b0s):
    BN = N_NODES // _NNB
    return pl.pallas_call(
        _lin0_body,
        grid=(NC, _NNB),
        in_specs=[
            pl.BlockSpec((BN, ATOM_IN), lambda h, i: (i, 0)),
            pl.BlockSpec((1, ATOM_IN, HALF), lambda h, i: (h, 0, 0)),
            pl.BlockSpec((1, 1, HALF), lambda h, i: (h, 0, 0)),
        ],
        out_specs=pl.BlockSpec((BN, HALF), lambda h, i: (h * _NNB + i, 0)),
        out_shape=jax.ShapeDtypeStruct((NC * N_NODES, HALF), jnp.float32),
    )(n_feat, W0s, b0s)


def _eterm_body(et_ref, w_ref, b_ref, o_ref):
    acc = jax.lax.dot_general(et_ref[...], w_ref[...],
                              (((0,), (0,)), ((), ())),
                              preferred_element_type=jnp.float32)
    o_ref[...] = acc + b_ref[...]


def _eterm(edge_attr_t, Wc_e, bc):
    BE = 16000
    return pl.pallas_call(
        _eterm_body,
        grid=(N_EDGES // BE,),
        in_specs=[
            pl.BlockSpec((BOND, BE), lambda i: (0, i)),
            pl.BlockSpec((BOND, HID), lambda i: (0, 0)),
            pl.BlockSpec((1, HID), lambda i: (0, 0)),
        ],
        out_specs=pl.BlockSpec((BE, HID), lambda i: (i, 0)),
        out_shape=jax.ShapeDtypeStruct((N_EDGES, HID), jnp.float32),
    )(edge_attr_t, Wc_e, bc.reshape(1, HID))


def _ab_body(x0_ref, x1_ref, w_ref, a_ref, b_ref):
    acc = jnp.dot(x0_ref[...], w_ref[0, 0], preferred_element_type=jnp.float32)
    acc = acc + jnp.dot(x1_ref[...], w_ref[0, 1],
                        preferred_element_type=jnp.float32)
    a_ref[...] = acc[:, :HALF]
    b_ref[...] = acc[:, HALF:]


def _ab(out, W4):
    BN = N_NODES // _NNB
    spec_flat = pl.BlockSpec((BN, HALF), lambda h, i: (h * _NNB + i, 0))
    return pl.pallas_call(
        _ab_body,
        grid=(NC, _NNB),
        in_specs=[
            pl.BlockSpec((BN, HALF), lambda h, i: (i, 0)),
            pl.BlockSpec((BN, HALF), lambda h, i: (_NNB + i, 0)),
            pl.BlockSpec((1, 2, HALF, HID), lambda h, i: (h, 0, 0, 0)),
        ],
        out_specs=[spec_flat, spec_flat],
        out_shape=[
            jax.ShapeDtypeStruct((NC * N_NODES, HALF), jnp.float32),
            jax.ShapeDtypeStruct((NC * N_NODES, HALF), jnp.float32),
        ],
    )(out, out, W4)


_HB = 5000
_HNB = NC * N_NODES // _HB


def _head_body(x_ref, w_ref, b_ref, o_ref, sacc):
    i = pl.program_id(0)

    @pl.when(i == 0)
    def _():
        sacc[...] = jnp.zeros_like(sacc)

    s = jnp.sum(x_ref[...], axis=0, keepdims=True)

    @pl.when(i < _HNB // 2)
    def _():
        sacc[:, :HALF] += s

    @pl.when(i >= _HNB // 2)
    def _():
        sacc[:, HALF:] += s

    @pl.when(i == _HNB - 1)
    def _():
        rep = jnp.maximum(sacc[...], 0.0)
        v = jnp.sum(rep * w_ref[...], keepdims=True) + b_ref[...]
        o_ref[...] = jnp.maximum(v, 0.0)


def _head(out, W2t, b2):
    return pl.pallas_call(
        _head_body,
        grid=(_HNB,),
        in_specs=[
            pl.BlockSpec((_HB, HALF), lambda i: (i, 0)),
            pl.BlockSpec((1, HID), lambda i: (0, 0)),
            pl.BlockSpec((1, 1), lambda i: (0, 0)),
        ],
        out_specs=pl.BlockSpec((1, 1), lambda i: (0, 0)),
        out_shape=jax.ShapeDtypeStruct((1, 1), jnp.float32),
        scratch_shapes=[pltpu.VMEM((1, HID), jnp.float32)],
    )(out, W2t, b2)


def _edge_body(a_hbm, b_hbm, et_hbm, idx_hbm, out_hbm,
               idxbuf, didxs, arow, brow, erow, mrow, zbuf, acc,
               idx_sem, gat_sem, sc_sem):
    cid = lax.axis_index("c")
    tid = lax.axis_index("s")
    e0 = tid * EPT
    coff = cid * N_NODES
    coff_c = cid * HALF

    zeros16 = jnp.zeros((L,), jnp.float32)

    def zrow(r, carry):
        for j in range(HALF // L):
            zbuf[r, pl.ds(j * L, L)] = zeros16
        return carry

    lax.fori_loop(0, DRN, zrow, 0)
    for k in range(KDC):
        m = tid + NS * k

        @pl.when(m < NDC)
        def _():
            pltpu.sync_copy(zbuf, acc.at[pl.ds(m * DRN, DRN)])

    plsc.subcore_barrier()

    def issue_idx(i, bi):
        off = e0 + i * C
        pltpu.async_copy(idx_hbm.at[cid, :, pl.ds(off, C)], idxbuf.at[bi],
                         idx_sem)

    def wait_idx(bi):
        pltpu.make_async_copy(idx_hbm.at[cid, :, pl.ds(0, C)], idxbuf.at[bi],
                              idx_sem).wait()

    def issue_gather(i, bi):
        off = e0 + i * C
        pltpu.async_copy(a_hbm.at[idxbuf.at[bi, 0]], arow.at[bi], gat_sem)
        pltpu.async_copy(b_hbm.at[idxbuf.at[bi, 1]], brow.at[bi], gat_sem)
        pltpu.async_copy(et_hbm.at[pl.ds(off, C), pl.ds(coff_c, HALF)],
                         erow.at[bi], gat_sem)

    def wait_gather(bi):
        pltpu.make_async_copy(a_hbm.at[idxbuf.at[bi, 0]], arow.at[bi],
                              gat_sem).wait()
        pltpu.make_async_copy(b_hbm.at[idxbuf.at[bi, 1]], brow.at[bi],
                              gat_sem).wait()
        pltpu.make_async_copy(et_hbm.at[pl.ds(0, C), pl.ds(coff_c, HALF)],
                              erow.at[bi], gat_sem).wait()

    def copy_scatter_idx(bi):
        for j in range(C // L):
            s = pl.ds(j * L, L)
            didxs[bi, s] = idxbuf[bi, 2, s]

    def issue_scatter(bi):
        pltpu.async_copy(mrow.at[bi], acc.at[didxs.at[bi]], sc_sem, add=True)

    def wait_scatter(bi):
        pltpu.make_async_copy(mrow.at[bi], acc.at[didxs.at[bi]], sc_sem).wait()

    CG = 16

    def compute(bi):
        def cgrp(g, carry):
            base = g * CG
            for r in range(CG):
                row = base + r
                for j in range(HALF // L):
                    s = pl.ds(j * L, L)
                    z = arow[bi, row, s] + brow[bi, row, s] + erow[bi, row, s]
                    lk = jnp.maximum(z, z * 0.01)
                    mrow[bi, row, s] = lk / (1.0 + jnp.exp(-z))
            return carry

        lax.fori_loop(0, C // CG, cgrp, 0)

    def chunk_step(i, bi):
        wait_gather(bi)

        @pl.when(i >= 2)
        def _():
            wait_scatter(bi)

        copy_scatter_idx(bi)

        @pl.when(i + 2 < NCHUNK)
        def _():
            issue_idx(i + 2, bi)

        compute(bi)
        issue_scatter(bi)

        @pl.when(i + 2 < NCHUNK)
        def _():
            wait_idx(bi)
            issue_gather(i + 2, bi)

    issue_idx(0, 0)
    issue_idx(1, 1)
    wait_idx(0)
    issue_gather(0, 0)
    wait_idx(1)
    issue_gather(1, 1)

    def pair_body(k, carry):
        i0 = 2 * k
        chunk_step(i0, 0)
        chunk_step(i0 + 1, 1)
        return carry

    lax.fori_loop(0, NCHUNK // 2, pair_body, 0)
    if NCHUNK % 2:
        chunk_step(NCHUNK - 1, 0)

    wait_scatter(1 - (NCHUNK % 2))
    wait_scatter(NCHUNK % 2)

    plsc.subcore_barrier()
    for k in range(KDC):
        m = tid + NS * k

        @pl.when(m < NDC)
        def _():
            pltpu.sync_copy(acc.at[pl.ds(m * DRN, DRN)],
                            out_hbm.at[pl.ds(coff + m * DRN, DRN)])


_edge_kernel = functools.partial(
    pl.kernel,
    out_type=jax.ShapeDtypeStruct((NC * N_NODES, HALF), jnp.float32),
    mesh=plsc.VectorSubcoreMesh(core_axis_name="c", subcore_axis_name="s",
                                num_cores=NC, num_subcores=NS),
    scratch_types=[
        pltpu.VMEM((2, 3, C), jnp.int32),
        pltpu.VMEM((2, C), jnp.int32),
        pltpu.VMEM((2, C, HALF), jnp.float32),
        pltpu.VMEM((2, C, HALF), jnp.float32),
        pltpu.VMEM((2, C, HALF), jnp.float32),
        pltpu.VMEM((2, C, HALF), jnp.float32),
        pltpu.VMEM((DRN, HALF), jnp.float32),
        pltpu.VMEM_SHARED((N_NODES, HALF), jnp.float32),
        pltpu.SemaphoreType.DMA,
        pltpu.SemaphoreType.DMA,
        pltpu.SemaphoreType.DMA,
    ],
    compiler_params=pltpu.CompilerParams(use_tc_tiling_on_sc=False,
                                         internal_scratch_in_bytes=0),
)(_edge_body)


def kernel(n_feat, edge_index, edge_attr, W0, b0, Wc, bc, W2, b2):
    src = edge_index[0].astype(jnp.int32)
    dst = edge_index[1].astype(jnp.int32)
    Ws = Wc[:HID]
    Wd = Wc[HID:2 * HID]
    We = Wc[2 * HID:]

    W0s = W0.reshape(ATOM_IN, NC, HALF).transpose(1, 0, 2)
    b0s = b0.reshape(NC, 1, HALF)
    idx_all = jnp.stack([
        jnp.stack([src + c * N_NODES, dst + c * N_NODES, dst])
        for c in range(NC)])
    W4 = jnp.stack([
        jnp.stack([
            jnp.concatenate([Ws[k * HALF:(k + 1) * HALF, h * HALF:(h + 1) * HALF],
                             Wd[k * HALF:(k + 1) * HALF, h * HALF:(h + 1) * HALF]],
                            axis=1)
            for k in range(NC)])
        for h in range(NC)])

    out = _lin0(n_feat, W0s, b0s)
    et = _eterm(edge_attr.T, We, bc)
    for _ in range(N_CONV):
        A, B = _ab(out, W4)
        out = _edge_kernel(A, B, et, idx_all)
    return _head(out, W2.reshape(1, HID), b2.reshape(1, 1))
